# Initial kernel scaffold; baseline (speedup 1.0000x reference)
import jax
import jax.numpy as jnp
from jax.experimental import pallas as pl


def kernel(x, keys):
    raise NotImplementedError("write your pallas kernel here")


# speedup vs baseline: 1.1787x; 1.1787x over previous
import functools

import jax
import jax.numpy as jnp
from jax import lax
from jax.experimental import pallas as pl
from jax.experimental.pallas import tpu as pltpu
from jax.experimental.pallas import tpu_sc as plsc

SIZE = 2048
BATCH = 16
ZDIM = 128
CERTAINTY = 10.0
MDEPTH = 11
NROWS = BATCH * SIZE
EXTRAS = 64


def _bucket_median(v):
    s = v.shape[-1]
    sv = jnp.sort(v, axis=-1)
    return sv[..., s // 2 - 1:s // 2 + 1].mean(axis=-1)


def _partition_rows(choices, depth, s):
    b = choices.shape[0]
    k = 2 ** depth
    sec = s // k
    half = sec // 2
    c = choices.reshape(b, k, sec).astype(jnp.int32)
    ones_before = jnp.cumsum(c, axis=-1) - c
    pos = jnp.broadcast_to(jnp.arange(sec)[None, None, :], c.shape)
    zeros_before = pos - ones_before
    local = jnp.where(c == 0, zeros_before, half + ones_before)
    sec_start = (jnp.arange(k) * sec)[None, :, None]
    return (local + sec_start).reshape(b, s)


def _keys_chain(keys):
    b, s = keys.shape
    cur_k = keys
    bidx = jnp.broadcast_to(jnp.arange(b)[:, None], (b, s))
    dest = jnp.broadcast_to(jnp.arange(s)[None, :], (b, s))
    for d in range(MDEPTH):
        nb = 2 ** d
        buckets = cur_k.reshape(b * nb, s // nb)
        piv = _bucket_median(buckets)
        piv = jnp.broadcast_to(piv[:, None], buckets.shape).reshape(b, s)
        offset = jax.nn.sigmoid((cur_k - piv) * CERTAINTY)
        choices = jnp.round(offset)
        rows = _partition_rows(choices, d, s)
        cur_k = jnp.zeros_like(cur_k).at[bidx, rows].add(cur_k)
        dest = jnp.take_along_axis(rows, dest, axis=1)
    return cur_k, dest


def _plan(dest):
    b, s = dest.shape
    n = b * s
    gdest = (dest + (jnp.arange(b) * s)[:, None]).reshape(-1)
    j = jnp.arange(n, dtype=jnp.int32)
    src = jnp.zeros((n,), jnp.int32).at[gdest].set(j)
    cnt = jnp.zeros((n,), jnp.int32).at[gdest].add(1)
    is_hole = cnt == 0
    is_extra = src[gdest] != j
    hole_rank = (jnp.cumsum(is_hole) - is_hole).astype(jnp.int32)
    ext_rank = (jnp.cumsum(is_extra) - is_extra).astype(jnp.int32)
    n_holes = jnp.sum(is_hole).astype(jnp.int32)
    slot_h = jnp.where(is_hole, hole_rank, EXTRAS)
    slot_e = jnp.minimum(jnp.where(is_extra, ext_rank + n_holes, EXTRAS), EXTRAS)
    tgt = (jnp.zeros((EXTRAS + 1,), jnp.int32)
           .at[slot_h].set(j, mode="drop")
           .at[slot_e].set(gdest.astype(jnp.int32), mode="drop"))[:EXTRAS]
    srce = (jnp.zeros((EXTRAS + 1,), jnp.int32)
            .at[slot_e].set(j, mode="drop"))[:EXTRAS]
    w = (jnp.zeros((EXTRAS + 1,), jnp.float32)
         .at[slot_h].set(-1.0, mode="drop")
         .at[slot_e].set(1.0, mode="drop"))[:EXTRAS]
    return src, tgt, srce, w


def _make_apply():
    info = plsc.get_sparse_core_info()
    nw = info.num_cores * info.num_subcores
    rows_per_w = NROWS // nw
    chunk = 512
    nchunks = rows_per_w // chunk
    mesh = plsc.VectorSubcoreMesh(core_axis_name="c", subcore_axis_name="s")

    iw = 128
    nsub = chunk // iw

    @functools.partial(
        pl.kernel, mesh=mesh,
        out_type=jax.ShapeDtypeStruct((NROWS, ZDIM), jnp.float32),
        compiler_params=pltpu.CompilerParams(needs_layout_passes=False),
        scratch_types=[
            pltpu.VMEM((rows_per_w // iw, iw), jnp.int32),
            pltpu.VMEM((chunk, ZDIM), jnp.float32),
            pltpu.VMEM((EXTRAS,), jnp.int32),
            pltpu.VMEM((EXTRAS,), jnp.int32),
            pltpu.VMEM((EXTRAS,), jnp.float32),
            pltpu.VMEM((EXTRAS, ZDIM), jnp.float32),
            pltpu.SemaphoreType.DMA,
        ],
    )
    def apply_kernel(x_hbm, src_hbm, tgt_hbm, esrc_hbm, ew_hbm, out_hbm,
                     idx_v, rows_v, tgt_v, esrc_v, ew_v, ext_v, sem):
        wid = lax.axis_index("s") * info.num_cores + lax.axis_index("c")
        base = pl.multiple_of(wid * rows_per_w, rows_per_w)
        pltpu.sync_copy(tgt_hbm, tgt_v)
        pltpu.sync_copy(esrc_hbm, esrc_v)
        pltpu.sync_copy(ew_hbm, ew_v)
        pltpu.async_copy(x_hbm.at[esrc_v], ext_v, sem).wait()
        pltpu.sync_copy(
            src_hbm.at[pl.ds(pl.multiple_of(base // iw, 8), rows_per_w // iw)],
            idx_v)
        for ci in range(nchunks):
            cbase = pl.multiple_of(base + ci * chunk, chunk)
            copies = [
                pltpu.async_copy(x_hbm.at[idx_v.at[ci * nsub + j]],
                                 rows_v.at[pl.ds(j * iw, iw)], sem)
                for j in range(nsub)
            ]
            for c in copies:
                c.wait()

            cb_vec = jnp.full((16,), cbase, jnp.int32)

            def fix(e, carry):
                e_idx = jnp.full((16,), e, jnp.int32)
                t_vec = plsc.load_gather(tgt_v, [e_idx])
                w_vec = plsc.load_gather(ew_v, [e_idx])
                loc = t_vec - cb_vec
                inside = (loc >= 0) & (loc < chunk)
                wm = jnp.where(inside, w_vec, jnp.zeros((16,), jnp.float32))
                locc = jnp.clip(loc, 0, chunk - 1)
                for kk in range(ZDIM // 16):
                    col = lax.iota(jnp.int32, 16) + (16 * kk)
                    v = plsc.load_gather(ext_v, [e_idx, col])
                    plsc.addupdate_scatter(rows_v, [locc, col], wm * v)
                return carry

            lax.fori_loop(0, EXTRAS, fix, 0)
            pltpu.sync_copy(rows_v, out_hbm.at[pl.ds(cbase, chunk)])

    return apply_kernel


_apply = _make_apply()


def kernel(x, keys):
    out_k, dest = _keys_chain(keys)
    src, tgt, srce, w = _plan(dest)
    x_flat = x.reshape(NROWS, ZDIM)
    out = _apply(x_flat, src.reshape(-1, 128), tgt, srce, w)
    return out.reshape(x.shape), out_k


# speedup vs baseline: 10.3393x; 8.7715x over previous
import functools

import numpy as np
import jax
import jax.numpy as jnp
from jax import lax
from jax.experimental import pallas as pl
from jax.experimental.pallas import tpu as pltpu
from jax.experimental.pallas import tpu_sc as plsc

SIZE = 2048
BATCH = 16
ZDIM = 128
MDEPTH = 11
NROWS = BATCH * SIZE
NV = SIZE // 16
ECAP = 16
EOUT = 128
ZSTAR = float(np.uint32(0x34B17219).view(np.float32))
IMIN = np.int32(-(2 ** 31))


def _make_chain():
    mesh = plsc.VectorSubcoreMesh(core_axis_name="c", subcore_axis_name="s")
    info = plsc.get_sparse_core_info()

    @functools.partial(
        pl.kernel, mesh=mesh,
        out_type=[
            jax.ShapeDtypeStruct((BATCH, SIZE), jnp.float32),
            jax.ShapeDtypeStruct((BATCH, SIZE), jnp.int32),
            jax.ShapeDtypeStruct((BATCH, EOUT), jnp.int32),
            jax.ShapeDtypeStruct((BATCH, EOUT), jnp.int32),
            jax.ShapeDtypeStruct((BATCH, EOUT), jnp.float32),
        ],
        compiler_params=pltpu.CompilerParams(needs_layout_passes=False),
        scratch_types=[
            pltpu.VMEM((SIZE,), jnp.float32),
            pltpu.VMEM((SIZE,), jnp.float32),
            pltpu.VMEM((SIZE,), jnp.float32),
            pltpu.VMEM((SIZE,), jnp.int32),
            pltpu.VMEM((SIZE,), jnp.int32),
            pltpu.VMEM((SIZE,), jnp.int32),
            pltpu.VMEM((SIZE,), jnp.int32),
            pltpu.VMEM((SIZE // 2,), jnp.int32),
            pltpu.VMEM((SIZE // 2,), jnp.float32),
            pltpu.VMEM((SIZE,), jnp.int32),
            pltpu.VMEM((SIZE,), jnp.int32),
            pltpu.VMEM((SIZE,), jnp.int32),
            pltpu.VMEM((SIZE,), jnp.int32),
            pltpu.VMEM((SIZE + 16,), jnp.int32),
            pltpu.VMEM((SIZE + 16,), jnp.int32),
            pltpu.VMEM((SIZE + 16,), jnp.float32),
            pltpu.SemaphoreType.DMA,
        ],
    )
    def chain(keys_hbm, keys_out, src_out, tgt_out, esrc_out, ew_out,
              kin, newk, S, U, U2, inclL, inclB, n1_buf, piv_buf, cbuf,
              dest, rows_buf, srcb, ltgt, lsrc, lw, sem):
        wid = lax.axis_index("s") * info.num_cores + lax.axis_index("c")
        IOTA = lax.iota(jnp.int32, 16)

        @pl.when(wid < BATCH)
        def _body():
            b = wid
            boff = b * SIZE
            pltpu.sync_copy(keys_hbm.at[b], kin)

            def seg_scan(v, carry, c, log2seg, segm1):
                base = v * 16
                lanes = IOTA + base
                incl_l = plsc.cumsum(c)
                inclL[pl.ds(base, 16)] = incl_l
                sstart = lanes - (lanes & segm1)
                sstart_local = jnp.maximum(sstart - base, 0)
                pidx = sstart_local - 1
                g = plsc.load_gather(inclL, [base + jnp.maximum(pidx, 0)])
                g = jnp.where(pidx >= 0, g, 0)
                incl_f = incl_l - g + jnp.where(sstart < base, carry, 0)
                inclB[pl.ds(base, 16)] = incl_f
                carry_new = plsc.load_gather(inclB, [jnp.full((16,), base + 15,
                                                             jnp.int32)])
                return incl_f, carry_new

            def radix_sort(log2seg):
                seg = lax.shift_left(jnp.int32(1), log2seg)
                segm1 = seg - 1

                def f2i_body(v, _):
                    kv = kin[pl.ds(v * 16, 16)]
                    bb = plsc.bitcast(kv, jnp.int32)
                    U[pl.ds(v * 16, 16)] = jnp.where(bb < 0, bb ^ -1, bb | IMIN)
                    return 0

                lax.fori_loop(0, NV, f2i_body, 0)

                def bit_pass(bsh, Usrc, Udst):
                    def sweep1(v, carry):
                        base = v * 16
                        lanes = IOTA + base
                        u = Usrc[pl.ds(base, 16)]
                        c = lax.shift_right_logical(u, bsh) & 1
                        incl_f, carry_new = seg_scan(v, carry, c, log2seg, segm1)
                        endm = ((lanes + 1) & segm1) == 0
                        segid = lax.shift_right_logical(lanes, log2seg)
                        plsc.store_scatter(n1_buf, [segid], incl_f, mask=endm)
                        return carry_new

                    lax.fori_loop(0, NV, sweep1, jnp.zeros((16,), jnp.int32))

                    def sweep2(v, _):
                        base = v * 16
                        lanes = IOTA + base
                        u = Usrc[pl.ds(base, 16)]
                        c = lax.shift_right_logical(u, bsh) & 1
                        incl_f = inclB[pl.ds(base, 16)]
                        segid = lax.shift_right_logical(lanes, log2seg)
                        sstart = lanes - (lanes & segm1)
                        n1 = plsc.load_gather(n1_buf, [segid])
                        ones_b = incl_f - c
                        zeros_b = (lanes - sstart) - ones_b
                        rows = sstart + jnp.where(
                            c == 1, (seg - n1) + ones_b, zeros_b)
                        plsc.store_scatter(Udst, [rows], u)
                        return 0

                    lax.fori_loop(0, NV, sweep2, 0)

                def bits_body(t, _):
                    bit_pass(2 * t, U, U2)
                    bit_pass(2 * t + 1, U2, U)
                    return 0

                lax.fori_loop(0, 16, bits_body, 0)

                def i2f_body(v, _):
                    u = U[pl.ds(v * 16, 16)]
                    bb = jnp.where(u < 0, u ^ IMIN, u ^ -1)
                    S[pl.ds(v * 16, 16)] = plsc.bitcast(bb, jnp.float32)
                    return 0

                lax.fori_loop(0, NV, i2f_body, 0)

            def init_body(v, _):
                dest[pl.ds(v * 16, 16)] = IOTA + v * 16
                return 0

            lax.fori_loop(0, NV, init_body, 0)

            def layer(d, flag):
                log2seg = 11 - d
                seg = lax.shift_left(jnp.int32(1), log2seg)
                segm1 = seg - 1
                half = lax.shift_right_logical(seg, 1)
                nb = lax.shift_right_logical(jnp.int32(SIZE), log2seg)

                @pl.when(flag > 0)
                def _():
                    radix_sort(log2seg)

                def piv_body(g, _):
                    j = IOTA + g * 16
                    jc = jnp.minimum(j, nb - 1)
                    i1 = jc * seg + half - 1
                    m1 = plsc.load_gather(S, [i1])
                    m2 = plsc.load_gather(S, [i1 + 1])
                    piv_buf[pl.ds(g * 16, 16)] = (m1 + m2) * 0.5
                    return 0

                lax.fori_loop(0, (nb + 15) >> 4, piv_body, 0)

                def csweep(v, carry):
                    cprev, flagv = carry
                    base = v * 16
                    lanes = IOTA + base
                    kv = kin[pl.ds(base, 16)]
                    pe = plsc.load_gather(
                        piv_buf, [lax.shift_right_logical(lanes, log2seg)])
                    t = (kv - pe) * 10.0
                    c = jnp.where(t >= ZSTAR, jnp.int32(1), jnp.int32(0))
                    cbuf[pl.ds(base, 16)] = c
                    incl_f, carry_new = seg_scan(v, cprev, c, log2seg, segm1)
                    endm = ((lanes + 1) & segm1) == 0
                    degen = endm & (incl_f != half)
                    flagv = flagv | jnp.where(degen, 1, 0)
                    return carry_new, flagv

                zero16 = jnp.zeros((16,), jnp.int32)
                _, flagv = lax.fori_loop(0, NV, csweep, (zero16, zero16))
                flag_new = jnp.where(jnp.max(flagv) > 0,
                                     jnp.int32(1), jnp.int32(0))

                def zsweep(v, _):
                    newk[pl.ds(v * 16, 16)] = jnp.zeros((16,), jnp.float32)
                    return 0

                lax.fori_loop(0, NV, zsweep, 0)

                def rsweep(v, _):
                    base = v * 16
                    lanes = IOTA + base
                    c = cbuf[pl.ds(base, 16)]
                    incl_f = inclB[pl.ds(base, 16)]
                    sstart = lanes - (lanes & segm1)
                    ones_b = incl_f - c
                    zeros_b = (lanes - sstart) - ones_b
                    rows = sstart + jnp.where(c == 1, half + ones_b, zeros_b)
                    rows_buf[pl.ds(base, 16)] = rows
                    kv = kin[pl.ds(base, 16)]
                    plsc.store_scatter(newk, [rows], kv, mask=c == 0)
                    return 0

                lax.fori_loop(0, NV, rsweep, 0)

                def asweep(v, _):
                    base = v * 16
                    c = cbuf[pl.ds(base, 16)]
                    rows = rows_buf[pl.ds(base, 16)]
                    kv = kin[pl.ds(base, 16)]
                    plsc.addupdate_scatter(newk, [rows], kv, mask=c == 1)
                    return 0

                lax.fori_loop(0, NV, asweep, 0)

                def copy_body(v, _):
                    kin[pl.ds(v * 16, 16)] = newk[pl.ds(v * 16, 16)]
                    dv = dest[pl.ds(v * 16, 16)]
                    dest[pl.ds(v * 16, 16)] = plsc.load_gather(rows_buf, [dv])
                    return 0

                lax.fori_loop(0, NV, copy_body, 0)
                return flag_new

            lax.fori_loop(0, MDEPTH, layer, jnp.int32(1))

            def sinit(v, _):
                srcb[pl.ds(v * 16, 16)] = jnp.full((16,), -1, jnp.int32)
                return 0

            lax.fori_loop(0, NV, sinit, 0)

            def sinv(v, _):
                dv = dest[pl.ds(v * 16, 16)]
                plsc.store_scatter(srcb, [dv], IOTA + v * 16)
                return 0

            lax.fori_loop(0, NV, sinv, 0)

            def linit(v, _):
                ltgt[pl.ds(v * 16, 16)] = jnp.zeros((16,), jnp.int32)
                lsrc[pl.ds(v * 16, 16)] = jnp.zeros((16,), jnp.int32)
                lw[pl.ds(v * 16, 16)] = jnp.zeros((16,), jnp.float32)
                return 0

            lax.fori_loop(0, NV + 1, linit, 0)

            def plan_sweep(v, off):
                base = v * 16
                lanes = IOTA + base
                rep = srcb[pl.ds(base, 16)]
                repc = jnp.clip(rep, 0, SIZE - 1)
                dchk = plsc.load_gather(dest, [repc])
                bad = (rep < 0) | (dchk != lanes)
                srcf = jnp.where(rep < 0, jnp.int32(0), repc)
                rows_buf[pl.ds(base, 16)] = srcf + boff
                plsc.store_compressed(ltgt.at[pl.ds(off, 16)],
                                      lanes + boff, mask=bad)
                plsc.store_compressed(lsrc.at[pl.ds(off, 16)],
                                      srcf + boff, mask=bad)
                plsc.store_compressed(lw.at[pl.ds(off, 16)],
                                      jnp.full((16,), -1.0, jnp.float32),
                                      mask=bad)
                off = off + jnp.max(plsc.all_reduce_population_count(bad))
                dv = dest[pl.ds(base, 16)]
                repi = plsc.load_gather(srcb, [dv])
                ex = repi != lanes
                plsc.store_compressed(ltgt.at[pl.ds(off, 16)], dv + boff, mask=ex)
                plsc.store_compressed(lsrc.at[pl.ds(off, 16)],
                                      lanes + boff, mask=ex)
                plsc.store_compressed(lw.at[pl.ds(off, 16)],
                                      jnp.full((16,), 1.0, jnp.float32),
                                      mask=ex)
                off = off + jnp.max(plsc.all_reduce_population_count(ex))
                return off

            lax.fori_loop(0, NV, plan_sweep, jnp.int32(0))

            pltpu.sync_copy(kin, keys_out.at[b])
            pltpu.sync_copy(rows_buf, src_out.at[b])
            pltpu.sync_copy(ltgt.at[pl.ds(0, EOUT)], tgt_out.at[b])
            pltpu.sync_copy(lsrc.at[pl.ds(0, EOUT)], esrc_out.at[b])
            pltpu.sync_copy(lw.at[pl.ds(0, EOUT)], ew_out.at[b])

    return chain


def _make_apply():
    info = plsc.get_sparse_core_info()
    nw = info.num_cores * info.num_subcores
    rows_per_w = NROWS // nw
    chunk = 512
    nchunks = rows_per_w // chunk
    iw = 128
    nsub = chunk // iw
    mesh = plsc.VectorSubcoreMesh(core_axis_name="c", subcore_axis_name="s")

    @functools.partial(
        pl.kernel, mesh=mesh,
        out_type=jax.ShapeDtypeStruct((NROWS, ZDIM), jnp.float32),
        compiler_params=pltpu.CompilerParams(needs_layout_passes=False),
        scratch_types=[
            pltpu.VMEM((rows_per_w // iw, iw), jnp.int32),
            pltpu.VMEM((chunk, ZDIM), jnp.float32),
            pltpu.VMEM((EOUT,), jnp.int32),
            pltpu.VMEM((EOUT,), jnp.int32),
            pltpu.VMEM((EOUT,), jnp.float32),
            pltpu.VMEM((ECAP, ZDIM), jnp.float32),
            pltpu.SemaphoreType.DMA,
        ],
    )
    def apply_kernel(x_hbm, src_hbm, tgt_hbm, esrc_hbm, ew_hbm, out_hbm,
                     idx_v, rows_v, tgt_v, esrc_v, ew_v, ext_v, sem):
        wid = lax.axis_index("s") * info.num_cores + lax.axis_index("c")
        base = pl.multiple_of(wid * rows_per_w, rows_per_w)
        bt = wid // (SIZE // rows_per_w)
        pltpu.sync_copy(tgt_hbm.at[bt], tgt_v)
        pltpu.sync_copy(esrc_hbm.at[bt], esrc_v)
        pltpu.sync_copy(ew_hbm.at[bt], ew_v)
        pltpu.async_copy(x_hbm.at[esrc_v.at[pl.ds(0, ECAP)]], ext_v, sem).wait()
        pltpu.sync_copy(
            src_hbm.at[pl.ds(pl.multiple_of(base // iw, 8), rows_per_w // iw)],
            idx_v)
        for ci in range(nchunks):
            cbase = pl.multiple_of(base + ci * chunk, chunk)
            copies = [
                pltpu.async_copy(x_hbm.at[idx_v.at[ci * nsub + j]],
                                 rows_v.at[pl.ds(j * iw, iw)], sem)
                for j in range(nsub)
            ]
            for c in copies:
                c.wait()

            cb_vec = jnp.full((16,), cbase, jnp.int32)

            def fix(e, carry):
                e_idx = jnp.full((16,), e, jnp.int32)
                t_vec = plsc.load_gather(tgt_v, [e_idx])
                w_vec = plsc.load_gather(ew_v, [e_idx])
                loc = t_vec - cb_vec
                inside = (loc >= 0) & (loc < chunk)
                wm = jnp.where(inside, w_vec, jnp.zeros((16,), jnp.float32))
                locc = jnp.clip(loc, 0, chunk - 1)
                for kk in range(ZDIM // 16):
                    col = lax.iota(jnp.int32, 16) + (16 * kk)
                    v = plsc.load_gather(ext_v, [e_idx, col])
                    plsc.addupdate_scatter(rows_v, [locc, col], wm * v)
                return carry

            lax.fori_loop(0, ECAP, fix, 0)
            pltpu.sync_copy(rows_v, out_hbm.at[pl.ds(cbase, chunk)])

    return apply_kernel


_chain = _make_chain()
_apply = _make_apply()


def kernel(x, keys):
    out_k, srcg, tgt, esrc, ew = _chain(keys)
    x_flat = x.reshape(NROWS, ZDIM)
    out = _apply(x_flat, srcg.reshape(NROWS // 128, 128), tgt, esrc, ew)
    return out.reshape(x.shape), out_k


# speedup vs baseline: 19.9486x; 1.9294x over previous
import functools

import numpy as np
import jax
import jax.numpy as jnp
from jax import lax
from jax.experimental import pallas as pl
from jax.experimental.pallas import tpu as pltpu
from jax.experimental.pallas import tpu_sc as plsc

SIZE = 2048
BATCH = 16
ZDIM = 128
MDEPTH = 11
NROWS = BATCH * SIZE
NV = SIZE // 16
ECAP = 16
EOUT = 128
ZSTAR = float(np.uint32(0x34B17219).view(np.float32))
IMIN = np.int32(-(2 ** 31))


def _make_chain():
    mesh = plsc.VectorSubcoreMesh(core_axis_name="c", subcore_axis_name="s")
    info = plsc.get_sparse_core_info()

    @functools.partial(
        pl.kernel, mesh=mesh,
        out_type=[
            jax.ShapeDtypeStruct((BATCH, SIZE), jnp.float32),
            jax.ShapeDtypeStruct((BATCH, SIZE), jnp.int32),
            jax.ShapeDtypeStruct((BATCH, EOUT), jnp.int32),
            jax.ShapeDtypeStruct((BATCH, EOUT), jnp.int32),
            jax.ShapeDtypeStruct((BATCH, EOUT), jnp.float32),
        ],
        compiler_params=pltpu.CompilerParams(needs_layout_passes=False),
        scratch_types=[
            pltpu.VMEM((SIZE,), jnp.float32),
            pltpu.VMEM((SIZE,), jnp.float32),
            pltpu.VMEM((SIZE,), jnp.float32),
            pltpu.VMEM((SIZE,), jnp.int32),
            pltpu.VMEM((SIZE,), jnp.int32),
            pltpu.VMEM((SIZE // 2,), jnp.float32),
            pltpu.VMEM((SIZE,), jnp.int32),
            pltpu.VMEM((SIZE,), jnp.int32),
            pltpu.VMEM((SIZE,), jnp.int32),
            pltpu.VMEM((SIZE,), jnp.int32),
            pltpu.VMEM((SIZE + 16,), jnp.int32),
            pltpu.VMEM((SIZE + 16,), jnp.int32),
            pltpu.VMEM((SIZE + 16,), jnp.float32),
            pltpu.SemaphoreType.DMA,
        ],
    )
    def chain(keys_hbm, keys_out, src_out, tgt_out, esrc_out, ew_out,
              kin, newk, S, inclL, inclB, piv_buf, cbuf,
              dest, rows_buf, srcb, ltgt, lsrc, lw, sem):
        wid = lax.axis_index("s") * info.num_cores + lax.axis_index("c")
        IOTA = lax.iota(jnp.int32, 16)

        @pl.when(wid < BATCH)
        def _body():
            b = wid
            boff = b * SIZE
            pltpu.sync_copy(keys_hbm.at[b], kin)

            def seg_scan(v, carry, c, log2seg, segm1):
                base = v * 16
                lanes = IOTA + base
                incl_l = plsc.cumsum(c)
                inclL[pl.ds(base, 16)] = incl_l
                sstart = lanes - (lanes & segm1)
                sstart_local = jnp.maximum(sstart - base, 0)
                pidx = sstart_local - 1
                g = plsc.load_gather(inclL, [base + jnp.maximum(pidx, 0)])
                g = jnp.where(pidx >= 0, g, 0)
                incl_f = incl_l - g + jnp.where(sstart < base, carry, 0)
                inclB[pl.ds(base, 16)] = incl_f
                carry_new = plsc.load_gather(inclB, [jnp.full((16,), base + 15,
                                                             jnp.int32)])
                return incl_f, carry_new

            def sort_blocks(log2seg):
                seg = lax.shift_left(jnp.int32(1), log2seg)

                @pl.when(log2seg >= 4)
                def _big():
                    def phase0(v, _):
                        S[pl.ds(v * 16, 16)] = jnp.sort(kin[pl.ds(v * 16, 16)])
                        return 0

                    lax.fori_loop(0, NV, phase0, 0)

                    def comparator(u_vr, w_vr):
                        a = S[pl.ds(u_vr * 16, 16)]
                        b = S[pl.ds(w_vr * 16, 16)]
                        br = lax.rev(b, (0,))
                        mn = jnp.minimum(a, br)
                        mx = jnp.maximum(a, br)
                        S[pl.ds(u_vr * 16, 16)] = jnp.sort(mn)
                        S[pl.ds(w_vr * 16, 16)] = jnp.sort(mx)

                    def level(j, _):
                        R = lax.shift_left(jnp.int32(1), j)

                        def mirror(p, _):
                            g = lax.shift_right_logical(p, j)
                            t = p & (R - 1)
                            comparator(g * 2 * R + t, g * 2 * R + 2 * R - 1 - t)
                            return 0

                        lax.fori_loop(0, 64, mirror, 0)

                        def stage(sd, _):
                            log2d = j - 1 - sd
                            D = lax.shift_left(jnp.int32(1), log2d)

                            def comp(p, _):
                                u_vr = (lax.shift_left(
                                    lax.shift_right_logical(p, log2d),
                                    log2d + 1)) | (p & (D - 1))
                                comparator(u_vr, u_vr + D)
                                return 0

                            lax.fori_loop(0, 64, comp, 0)
                            return 0

                        lax.fori_loop(0, j, stage, 0)
                        return 0

                    lax.fori_loop(0, log2seg - 4, level, 0)

                @pl.when(log2seg < 4)
                def _small():
                    def vloop(v, _):
                        base = v * 16

                        def kloop(kk, xc):
                            k = lax.shift_left(jnp.int32(1), kk)

                            def jloop(sd, x2):
                                jv = lax.shift_left(jnp.int32(1),
                                                    kk - 1 - sd)
                                newk[pl.ds(base, 16)] = x2
                                p = plsc.load_gather(
                                    newk, [base + (IOTA ^ jv)])
                                lowlane = (IOTA & jv) == 0
                                asc = ((IOTA & k) == 0) | (k == seg)
                                keep_min = lowlane == asc
                                return jnp.where(keep_min,
                                                 jnp.minimum(x2, p),
                                                 jnp.maximum(x2, p))

                            return lax.fori_loop(0, kk, jloop, xc)

                        xs = lax.fori_loop(1, log2seg + 1, kloop,
                                           kin[pl.ds(base, 16)])
                        S[pl.ds(base, 16)] = xs
                        return 0

                    lax.fori_loop(0, NV, vloop, 0)

            def init_body(v, _):
                dest[pl.ds(v * 16, 16)] = IOTA + v * 16
                return 0

            lax.fori_loop(0, NV, init_body, 0)

            def layer(d, flag):
                log2seg = 11 - d
                seg = lax.shift_left(jnp.int32(1), log2seg)
                segm1 = seg - 1
                half = lax.shift_right_logical(seg, 1)
                nb = lax.shift_right_logical(jnp.int32(SIZE), log2seg)

                @pl.when(flag > 0)
                def _():
                    sort_blocks(log2seg)

                def piv_body(g, _):
                    j = IOTA + g * 16
                    jc = jnp.minimum(j, nb - 1)
                    i1 = jc * seg + half - 1
                    m1 = plsc.load_gather(S, [i1])
                    m2 = plsc.load_gather(S, [i1 + 1])
                    piv_buf[pl.ds(g * 16, 16)] = (m1 + m2) * 0.5
                    return 0

                lax.fori_loop(0, (nb + 15) >> 4, piv_body, 0)

                def csweep(v, carry):
                    cprev, flagv = carry
                    base = v * 16
                    lanes = IOTA + base
                    kv = kin[pl.ds(base, 16)]
                    pe = plsc.load_gather(
                        piv_buf, [lax.shift_right_logical(lanes, log2seg)])
                    t = (kv - pe) * 10.0
                    c = jnp.where(t >= ZSTAR, jnp.int32(1), jnp.int32(0))
                    cbuf[pl.ds(base, 16)] = c
                    incl_f, carry_new = seg_scan(v, cprev, c, log2seg, segm1)
                    endm = ((lanes + 1) & segm1) == 0
                    degen = endm & (incl_f != half)
                    flagv = flagv | jnp.where(degen, 1, 0)
                    return carry_new, flagv

                zero16 = jnp.zeros((16,), jnp.int32)
                _, flagv = lax.fori_loop(0, NV, csweep, (zero16, zero16))
                flag_new = jnp.where(jnp.max(flagv) > 0,
                                     jnp.int32(1), jnp.int32(0))

                def zsweep(v, _):
                    newk[pl.ds(v * 16, 16)] = jnp.zeros((16,), jnp.float32)
                    return 0

                lax.fori_loop(0, NV, zsweep, 0)

                def rsweep(v, _):
                    base = v * 16
                    lanes = IOTA + base
                    c = cbuf[pl.ds(base, 16)]
                    incl_f = inclB[pl.ds(base, 16)]
                    sstart = lanes - (lanes & segm1)
                    ones_b = incl_f - c
                    zeros_b = (lanes - sstart) - ones_b
                    rows = sstart + jnp.where(c == 1, half + ones_b, zeros_b)
                    rows_buf[pl.ds(base, 16)] = rows
                    kv = kin[pl.ds(base, 16)]
                    plsc.store_scatter(newk, [rows], kv, mask=c == 0)
                    return 0

                lax.fori_loop(0, NV, rsweep, 0)

                def asweep(v, _):
                    base = v * 16
                    c = cbuf[pl.ds(base, 16)]
                    rows = rows_buf[pl.ds(base, 16)]
                    kv = kin[pl.ds(base, 16)]
                    plsc.addupdate_scatter(newk, [rows], kv, mask=c == 1)
                    return 0

                lax.fori_loop(0, NV, asweep, 0)

                def copy_body(v, _):
                    kin[pl.ds(v * 16, 16)] = newk[pl.ds(v * 16, 16)]
                    dv = dest[pl.ds(v * 16, 16)]
                    dest[pl.ds(v * 16, 16)] = plsc.load_gather(rows_buf, [dv])
                    return 0

                lax.fori_loop(0, NV, copy_body, 0)
                return flag_new

            lax.fori_loop(0, MDEPTH, layer, jnp.int32(1))

            def sinit(v, _):
                srcb[pl.ds(v * 16, 16)] = jnp.full((16,), -1, jnp.int32)
                return 0

            lax.fori_loop(0, NV, sinit, 0)

            def sinv(v, _):
                dv = dest[pl.ds(v * 16, 16)]
                plsc.store_scatter(srcb, [dv], IOTA + v * 16)
                return 0

            lax.fori_loop(0, NV, sinv, 0)

            def linit(v, _):
                ltgt[pl.ds(v * 16, 16)] = jnp.zeros((16,), jnp.int32)
                lsrc[pl.ds(v * 16, 16)] = jnp.zeros((16,), jnp.int32)
                lw[pl.ds(v * 16, 16)] = jnp.zeros((16,), jnp.float32)
                return 0

            lax.fori_loop(0, NV + 1, linit, 0)

            def plan_sweep(v, off):
                base = v * 16
                lanes = IOTA + base
                rep = srcb[pl.ds(base, 16)]
                repc = jnp.clip(rep, 0, SIZE - 1)
                dchk = plsc.load_gather(dest, [repc])
                bad = (rep < 0) | (dchk != lanes)
                srcf = jnp.where(rep < 0, jnp.int32(0), repc)
                rows_buf[pl.ds(base, 16)] = srcf + boff
                plsc.store_compressed(ltgt.at[pl.ds(off, 16)],
                                      lanes + boff, mask=bad)
                plsc.store_compressed(lsrc.at[pl.ds(off, 16)],
                                      srcf + boff, mask=bad)
                plsc.store_compressed(lw.at[pl.ds(off, 16)],
                                      jnp.full((16,), -1.0, jnp.float32),
                                      mask=bad)
                off = off + jnp.max(plsc.all_reduce_population_count(bad))
                dv = dest[pl.ds(base, 16)]
                repi = plsc.load_gather(srcb, [dv])
                ex = repi != lanes
                plsc.store_compressed(ltgt.at[pl.ds(off, 16)], dv + boff, mask=ex)
                plsc.store_compressed(lsrc.at[pl.ds(off, 16)],
                                      lanes + boff, mask=ex)
                plsc.store_compressed(lw.at[pl.ds(off, 16)],
                                      jnp.full((16,), 1.0, jnp.float32),
                                      mask=ex)
                off = off + jnp.max(plsc.all_reduce_population_count(ex))
                return off

            lax.fori_loop(0, NV, plan_sweep, jnp.int32(0))

            pltpu.sync_copy(kin, keys_out.at[b])
            pltpu.sync_copy(rows_buf, src_out.at[b])
            pltpu.sync_copy(ltgt.at[pl.ds(0, EOUT)], tgt_out.at[b])
            pltpu.sync_copy(lsrc.at[pl.ds(0, EOUT)], esrc_out.at[b])
            pltpu.sync_copy(lw.at[pl.ds(0, EOUT)], ew_out.at[b])

    return chain


def _make_apply():
    info = plsc.get_sparse_core_info()
    nw = info.num_cores * info.num_subcores
    rows_per_w = NROWS // nw
    chunk = 512
    nchunks = rows_per_w // chunk
    iw = 128
    nsub = chunk // iw
    mesh = plsc.VectorSubcoreMesh(core_axis_name="c", subcore_axis_name="s")

    @functools.partial(
        pl.kernel, mesh=mesh,
        out_type=jax.ShapeDtypeStruct((NROWS, ZDIM), jnp.float32),
        compiler_params=pltpu.CompilerParams(needs_layout_passes=False),
        scratch_types=[
            pltpu.VMEM((rows_per_w // iw, iw), jnp.int32),
            pltpu.VMEM((chunk, ZDIM), jnp.float32),
            pltpu.VMEM((EOUT,), jnp.int32),
            pltpu.VMEM((EOUT,), jnp.int32),
            pltpu.VMEM((EOUT,), jnp.float32),
            pltpu.VMEM((ECAP, ZDIM), jnp.float32),
            pltpu.SemaphoreType.DMA,
        ],
    )
    def apply_kernel(x_hbm, src_hbm, tgt_hbm, esrc_hbm, ew_hbm, out_hbm,
                     idx_v, rows_v, tgt_v, esrc_v, ew_v, ext_v, sem):
        wid = lax.axis_index("s") * info.num_cores + lax.axis_index("c")
        base = pl.multiple_of(wid * rows_per_w, rows_per_w)
        bt = wid // (SIZE // rows_per_w)
        pltpu.sync_copy(tgt_hbm.at[bt], tgt_v)
        pltpu.sync_copy(esrc_hbm.at[bt], esrc_v)
        pltpu.sync_copy(ew_hbm.at[bt], ew_v)
        pltpu.async_copy(x_hbm.at[esrc_v.at[pl.ds(0, ECAP)]], ext_v, sem).wait()
        pltpu.sync_copy(
            src_hbm.at[pl.ds(pl.multiple_of(base // iw, 8), rows_per_w // iw)],
            idx_v)
        for ci in range(nchunks):
            cbase = pl.multiple_of(base + ci * chunk, chunk)
            copies = [
                pltpu.async_copy(x_hbm.at[idx_v.at[ci * nsub + j]],
                                 rows_v.at[pl.ds(j * iw, iw)], sem)
                for j in range(nsub)
            ]
            for c in copies:
                c.wait()

            cb_vec = jnp.full((16,), cbase, jnp.int32)

            def fix(e, carry):
                e_idx = jnp.full((16,), e, jnp.int32)
                t_vec = plsc.load_gather(tgt_v, [e_idx])
                w_vec = plsc.load_gather(ew_v, [e_idx])
                loc = t_vec - cb_vec
                inside = (loc >= 0) & (loc < chunk)
                wm = jnp.where(inside, w_vec, jnp.zeros((16,), jnp.float32))
                locc = jnp.clip(loc, 0, chunk - 1)
                for kk in range(ZDIM // 16):
                    col = lax.iota(jnp.int32, 16) + (16 * kk)
                    v = plsc.load_gather(ext_v, [e_idx, col])
                    plsc.addupdate_scatter(rows_v, [locc, col], wm * v)
                return carry

            lax.fori_loop(0, ECAP, fix, 0)
            pltpu.sync_copy(rows_v, out_hbm.at[pl.ds(cbase, chunk)])

    return apply_kernel


_chain = _make_chain()
_apply = _make_apply()


def kernel(x, keys):
    out_k, srcg, tgt, esrc, ew = _chain(keys)
    x_flat = x.reshape(NROWS, ZDIM)
    out = _apply(x_flat, srcg.reshape(NROWS // 128, 128), tgt, esrc, ew)
    return out.reshape(x.shape), out_k


# speedup vs baseline: 22.6061x; 1.1332x over previous
import functools

import numpy as np
import jax
import jax.numpy as jnp
from jax import lax
from jax.experimental import pallas as pl
from jax.experimental.pallas import tpu as pltpu
from jax.experimental.pallas import tpu_sc as plsc

SIZE = 2048
BATCH = 16
ZDIM = 128
MDEPTH = 11
NROWS = BATCH * SIZE
NV = SIZE // 16
ECAP = 16
EOUT = 128
ZSTAR = float(np.uint32(0x34B17219).view(np.float32))
IMIN = np.int32(-(2 ** 31))


def _make_chain():
    mesh = plsc.VectorSubcoreMesh(core_axis_name="c", subcore_axis_name="s")
    info = plsc.get_sparse_core_info()

    @functools.partial(
        pl.kernel, mesh=mesh,
        out_type=[
            jax.ShapeDtypeStruct((BATCH, SIZE), jnp.float32),
            jax.ShapeDtypeStruct((BATCH, SIZE), jnp.int32),
            jax.ShapeDtypeStruct((BATCH, EOUT), jnp.int32),
            jax.ShapeDtypeStruct((BATCH, EOUT), jnp.int32),
            jax.ShapeDtypeStruct((BATCH, EOUT), jnp.float32),
        ],
        compiler_params=pltpu.CompilerParams(needs_layout_passes=False),
        scratch_types=[
            pltpu.VMEM((SIZE,), jnp.float32),
            pltpu.VMEM((SIZE,), jnp.float32),
            pltpu.VMEM((SIZE,), jnp.float32),
            pltpu.VMEM((SIZE,), jnp.int32),
            pltpu.VMEM((SIZE,), jnp.int32),
            pltpu.VMEM((SIZE // 2,), jnp.float32),
            pltpu.VMEM((SIZE,), jnp.int32),
            pltpu.VMEM((SIZE,), jnp.int32),
            pltpu.VMEM((SIZE,), jnp.int32),
            pltpu.VMEM((SIZE,), jnp.int32),
            pltpu.VMEM((SIZE + 16,), jnp.int32),
            pltpu.VMEM((SIZE + 16,), jnp.int32),
            pltpu.VMEM((SIZE + 16,), jnp.float32),
            pltpu.SemaphoreType.DMA,
        ],
    )
    def chain(keys_hbm, keys_out, src_out, tgt_out, esrc_out, ew_out,
              kin, newk, S, inclL, inclB, piv_buf, cbuf,
              dest, rows_buf, srcb, ltgt, lsrc, lw, sem):
        wid = lax.axis_index("s") * info.num_cores + lax.axis_index("c")
        IOTA = lax.iota(jnp.int32, 16)

        @pl.when(wid < BATCH)
        def _body():
            b = wid
            boff = b * SIZE
            pltpu.sync_copy(keys_hbm.at[b], kin)

            def seg_scan(v, carry, c, log2seg, segm1):
                base = v * 16
                lanes = IOTA + base
                incl_l = plsc.cumsum(c)
                sstart = lanes - (lanes & segm1)
                sstart_local = jnp.maximum(sstart - base, 0)
                pidx = sstart_local - 1
                g = incl_l[jnp.maximum(pidx, 0)]
                g = jnp.where(pidx >= 0, g, 0)
                incl_f = incl_l - g + jnp.where(sstart < base, carry, 0)
                inclB[pl.ds(base, 16)] = incl_f
                carry_new = incl_f[jnp.full((16,), 15, jnp.int32)]
                return incl_f, carry_new

            def sort_blocks(log2seg):
                seg = lax.shift_left(jnp.int32(1), log2seg)

                @pl.when(log2seg >= 4)
                def _big():
                    def phase0(v, _):
                        S[pl.ds(v * 16, 16)] = jnp.sort(kin[pl.ds(v * 16, 16)])
                        return 0

                    lax.fori_loop(0, NV, phase0, 0)

                    def comparator(u_vr, w_vr):
                        a = S[pl.ds(u_vr * 16, 16)]
                        b = S[pl.ds(w_vr * 16, 16)]
                        br = lax.rev(b, (0,))
                        mn = jnp.minimum(a, br)
                        mx = jnp.maximum(a, br)
                        S[pl.ds(u_vr * 16, 16)] = jnp.sort(mn)
                        S[pl.ds(w_vr * 16, 16)] = jnp.sort(mx)

                    def level(j, _):
                        R = lax.shift_left(jnp.int32(1), j)

                        def mirror(p, _):
                            g = lax.shift_right_logical(p, j)
                            t = p & (R - 1)
                            comparator(g * 2 * R + t, g * 2 * R + 2 * R - 1 - t)
                            return 0

                        lax.fori_loop(0, 64, mirror, 0)

                        def stage(sd, _):
                            log2d = j - 1 - sd
                            D = lax.shift_left(jnp.int32(1), log2d)

                            def comp(p, _):
                                u_vr = (lax.shift_left(
                                    lax.shift_right_logical(p, log2d),
                                    log2d + 1)) | (p & (D - 1))
                                comparator(u_vr, u_vr + D)
                                return 0

                            lax.fori_loop(0, 64, comp, 0)
                            return 0

                        lax.fori_loop(0, j, stage, 0)
                        return 0

                    lax.fori_loop(0, log2seg - 4, level, 0)

                @pl.when(log2seg < 4)
                def _small():
                    def vloop(v, _):
                        base = v * 16

                        def kloop(kk, xc):
                            k = lax.shift_left(jnp.int32(1), kk)

                            def jloop(sd, x2):
                                jv = lax.shift_left(jnp.int32(1),
                                                    kk - 1 - sd)
                                newk[pl.ds(base, 16)] = x2
                                p = plsc.load_gather(
                                    newk, [base + (IOTA ^ jv)])
                                lowlane = (IOTA & jv) == 0
                                asc = ((IOTA & k) == 0) | (k == seg)
                                keep_min = lowlane == asc
                                return jnp.where(keep_min,
                                                 jnp.minimum(x2, p),
                                                 jnp.maximum(x2, p))

                            return lax.fori_loop(0, kk, jloop, xc)

                        xs = lax.fori_loop(1, log2seg + 1, kloop,
                                           kin[pl.ds(base, 16)])
                        S[pl.ds(base, 16)] = xs
                        newk[pl.ds(base, 16)] = jnp.zeros((16,), jnp.float32)
                        return 0

                    lax.fori_loop(0, NV, vloop, 0)

            def init_body(v, _):
                dest[pl.ds(v * 16, 16)] = IOTA + v * 16
                newk[pl.ds(v * 16, 16)] = jnp.zeros((16,), jnp.float32)
                return 0

            lax.fori_loop(0, NV, init_body, 0)

            def layer(d, flag):
                log2seg = 11 - d
                seg = lax.shift_left(jnp.int32(1), log2seg)
                segm1 = seg - 1
                half = lax.shift_right_logical(seg, 1)
                nb = lax.shift_right_logical(jnp.int32(SIZE), log2seg)

                @pl.when(flag > 0)
                def _():
                    sort_blocks(log2seg)

                def piv_body(g, _):
                    j = IOTA + g * 16
                    jc = jnp.minimum(j, nb - 1)
                    i1 = jc * seg + half - 1
                    m1 = plsc.load_gather(S, [i1])
                    m2 = plsc.load_gather(S, [i1 + 1])
                    piv_buf[pl.ds(g * 16, 16)] = (m1 + m2) * 0.5
                    return 0

                lax.fori_loop(0, (nb + 15) >> 4, piv_body, 0)

                def csweep(v, carry):
                    cprev, flagv = carry
                    base = v * 16
                    lanes = IOTA + base
                    kv = kin[pl.ds(base, 16)]
                    pe = plsc.load_gather(
                        piv_buf, [lax.shift_right_logical(lanes, log2seg)])
                    t = (kv - pe) * 10.0
                    c = jnp.where(t >= ZSTAR, jnp.int32(1), jnp.int32(0))
                    cbuf[pl.ds(base, 16)] = c
                    incl_f, carry_new = seg_scan(v, cprev, c, log2seg, segm1)
                    endm = ((lanes + 1) & segm1) == 0
                    degen = endm & (incl_f != half)
                    flagv = flagv | jnp.where(degen, 1, 0)
                    return carry_new, flagv

                zero16 = jnp.zeros((16,), jnp.int32)
                _, flagv = lax.fori_loop(0, NV, csweep, (zero16, zero16))
                flag_new = jnp.where(jnp.max(flagv) > 0,
                                     jnp.int32(1), jnp.int32(0))

                def rsweep(v, _):
                    base = v * 16
                    lanes = IOTA + base
                    c = cbuf[pl.ds(base, 16)]
                    incl_f = inclB[pl.ds(base, 16)]
                    sstart = lanes - (lanes & segm1)
                    ones_b = incl_f - c
                    zeros_b = (lanes - sstart) - ones_b
                    rows = sstart + jnp.where(c == 1, half + ones_b, zeros_b)
                    rows_buf[pl.ds(base, 16)] = rows
                    kv = kin[pl.ds(base, 16)]
                    plsc.addupdate_scatter(newk, [rows], kv, mask=c == 0)
                    plsc.addupdate_scatter(newk, [rows], kv, mask=c == 1)
                    return 0

                lax.fori_loop(0, NV, rsweep, 0)

                def copy_body(v, _):
                    kin[pl.ds(v * 16, 16)] = newk[pl.ds(v * 16, 16)]
                    newk[pl.ds(v * 16, 16)] = jnp.zeros((16,), jnp.float32)
                    dv = dest[pl.ds(v * 16, 16)]
                    dest[pl.ds(v * 16, 16)] = plsc.load_gather(rows_buf, [dv])
                    return 0

                lax.fori_loop(0, NV, copy_body, 0)
                return flag_new

            lax.fori_loop(0, MDEPTH, layer, jnp.int32(1))

            def sinit(v, _):
                srcb[pl.ds(v * 16, 16)] = jnp.full((16,), -1, jnp.int32)
                return 0

            lax.fori_loop(0, NV, sinit, 0)

            def sinv(v, _):
                dv = dest[pl.ds(v * 16, 16)]
                plsc.store_scatter(srcb, [dv], IOTA + v * 16)
                return 0

            lax.fori_loop(0, NV, sinv, 0)

            def linit(v, _):
                ltgt[pl.ds(v * 16, 16)] = jnp.zeros((16,), jnp.int32)
                lsrc[pl.ds(v * 16, 16)] = jnp.zeros((16,), jnp.int32)
                lw[pl.ds(v * 16, 16)] = jnp.zeros((16,), jnp.float32)
                return 0

            lax.fori_loop(0, NV + 1, linit, 0)

            def plan_sweep(v, off):
                base = v * 16
                lanes = IOTA + base
                rep = srcb[pl.ds(base, 16)]
                repc = jnp.clip(rep, 0, SIZE - 1)
                dchk = plsc.load_gather(dest, [repc])
                bad = (rep < 0) | (dchk != lanes)
                srcf = jnp.where(rep < 0, jnp.int32(0), repc)
                rows_buf[pl.ds(base, 16)] = srcf + boff
                plsc.store_compressed(ltgt.at[pl.ds(off, 16)],
                                      lanes + boff, mask=bad)
                plsc.store_compressed(lsrc.at[pl.ds(off, 16)],
                                      srcf + boff, mask=bad)
                plsc.store_compressed(lw.at[pl.ds(off, 16)],
                                      jnp.full((16,), -1.0, jnp.float32),
                                      mask=bad)
                off = off + jnp.max(plsc.all_reduce_population_count(bad))
                dv = dest[pl.ds(base, 16)]
                repi = plsc.load_gather(srcb, [dv])
                ex = repi != lanes
                plsc.store_compressed(ltgt.at[pl.ds(off, 16)], dv + boff, mask=ex)
                plsc.store_compressed(lsrc.at[pl.ds(off, 16)],
                                      lanes + boff, mask=ex)
                plsc.store_compressed(lw.at[pl.ds(off, 16)],
                                      jnp.full((16,), 1.0, jnp.float32),
                                      mask=ex)
                off = off + jnp.max(plsc.all_reduce_population_count(ex))
                return off

            lax.fori_loop(0, NV, plan_sweep, jnp.int32(0))

            pltpu.sync_copy(kin, keys_out.at[b])
            pltpu.sync_copy(rows_buf, src_out.at[b])
            pltpu.sync_copy(ltgt.at[pl.ds(0, EOUT)], tgt_out.at[b])
            pltpu.sync_copy(lsrc.at[pl.ds(0, EOUT)], esrc_out.at[b])
            pltpu.sync_copy(lw.at[pl.ds(0, EOUT)], ew_out.at[b])

    return chain


def _make_apply():
    info = plsc.get_sparse_core_info()
    nw = info.num_cores * info.num_subcores
    rows_per_w = NROWS // nw
    chunk = 512
    nchunks = rows_per_w // chunk
    iw = 128
    nsub = chunk // iw
    mesh = plsc.VectorSubcoreMesh(core_axis_name="c", subcore_axis_name="s")

    @functools.partial(
        pl.kernel, mesh=mesh,
        out_type=jax.ShapeDtypeStruct((NROWS, ZDIM), jnp.float32),
        compiler_params=pltpu.CompilerParams(needs_layout_passes=False),
        scratch_types=[
            pltpu.VMEM((rows_per_w // iw, iw), jnp.int32),
            pltpu.VMEM((chunk, ZDIM), jnp.float32),
            pltpu.VMEM((EOUT,), jnp.int32),
            pltpu.VMEM((EOUT,), jnp.int32),
            pltpu.VMEM((EOUT,), jnp.float32),
            pltpu.VMEM((ECAP, ZDIM), jnp.float32),
            pltpu.SemaphoreType.DMA,
        ],
    )
    def apply_kernel(x_hbm, src_hbm, tgt_hbm, esrc_hbm, ew_hbm, out_hbm,
                     idx_v, rows_v, tgt_v, esrc_v, ew_v, ext_v, sem):
        wid = lax.axis_index("s") * info.num_cores + lax.axis_index("c")
        base = pl.multiple_of(wid * rows_per_w, rows_per_w)
        bt = wid // (SIZE // rows_per_w)
        pltpu.sync_copy(tgt_hbm.at[bt], tgt_v)
        pltpu.sync_copy(esrc_hbm.at[bt], esrc_v)
        pltpu.sync_copy(ew_hbm.at[bt], ew_v)
        pltpu.async_copy(x_hbm.at[esrc_v.at[pl.ds(0, ECAP)]], ext_v, sem).wait()
        pltpu.sync_copy(
            src_hbm.at[pl.ds(pl.multiple_of(base // iw, 8), rows_per_w // iw)],
            idx_v)
        for ci in range(nchunks):
            cbase = pl.multiple_of(base + ci * chunk, chunk)
            copies = [
                pltpu.async_copy(x_hbm.at[idx_v.at[ci * nsub + j]],
                                 rows_v.at[pl.ds(j * iw, iw)], sem)
                for j in range(nsub)
            ]
            for c in copies:
                c.wait()

            cb_vec = jnp.full((16,), cbase, jnp.int32)

            def fix(e, carry):
                e_idx = jnp.full((16,), e, jnp.int32)
                t_vec = plsc.load_gather(tgt_v, [e_idx])
                w_vec = plsc.load_gather(ew_v, [e_idx])
                loc = t_vec - cb_vec
                inside = (loc >= 0) & (loc < chunk)
                wm = jnp.where(inside, w_vec, jnp.zeros((16,), jnp.float32))
                locc = jnp.clip(loc, 0, chunk - 1)
                for kk in range(ZDIM // 16):
                    col = lax.iota(jnp.int32, 16) + (16 * kk)
                    v = plsc.load_gather(ext_v, [e_idx, col])
                    plsc.addupdate_scatter(rows_v, [locc, col], wm * v)
                return carry

            lax.fori_loop(0, ECAP, fix, 0)
            pltpu.sync_copy(rows_v, out_hbm.at[pl.ds(cbase, chunk)])

    return apply_kernel


_chain = _make_chain()
_apply = _make_apply()


def kernel(x, keys):
    out_k, srcg, tgt, esrc, ew = _chain(keys)
    x_flat = x.reshape(NROWS, ZDIM)
    out = _apply(x_flat, srcg.reshape(NROWS // 128, 128), tgt, esrc, ew)
    return out.reshape(x.shape), out_k


# speedup vs baseline: 22.6603x; 1.0024x over previous
import functools

import numpy as np
import jax
import jax.numpy as jnp
from jax import lax
from jax.experimental import pallas as pl
from jax.experimental.pallas import tpu as pltpu
from jax.experimental.pallas import tpu_sc as plsc

SIZE = 2048
BATCH = 16
ZDIM = 128
MDEPTH = 11
NROWS = BATCH * SIZE
NV = SIZE // 16
ECAP = 16
EOUT = 128
ZSTAR = float(np.uint32(0x34B17219).view(np.float32))
IMIN = np.int32(-(2 ** 31))


def _make_chain():
    mesh = plsc.VectorSubcoreMesh(core_axis_name="c", subcore_axis_name="s")
    info = plsc.get_sparse_core_info()

    @functools.partial(
        pl.kernel, mesh=mesh,
        out_type=[
            jax.ShapeDtypeStruct((BATCH, SIZE), jnp.float32),
            jax.ShapeDtypeStruct((BATCH, SIZE), jnp.int32),
            jax.ShapeDtypeStruct((BATCH, EOUT), jnp.int32),
            jax.ShapeDtypeStruct((BATCH, EOUT), jnp.int32),
            jax.ShapeDtypeStruct((BATCH, EOUT), jnp.float32),
        ],
        compiler_params=pltpu.CompilerParams(needs_layout_passes=False),
        scratch_types=[
            pltpu.VMEM((SIZE,), jnp.float32),
            pltpu.VMEM((SIZE,), jnp.float32),
            pltpu.VMEM((SIZE,), jnp.float32),
            pltpu.VMEM((SIZE,), jnp.int32),
            pltpu.VMEM((SIZE,), jnp.int32),
            pltpu.VMEM((SIZE // 2,), jnp.float32),
            pltpu.VMEM((SIZE,), jnp.int32),
            pltpu.VMEM((SIZE,), jnp.int32),
            pltpu.VMEM((SIZE,), jnp.int32),
            pltpu.VMEM((SIZE,), jnp.int32),
            pltpu.VMEM((SIZE + 16,), jnp.int32),
            pltpu.VMEM((SIZE + 16,), jnp.int32),
            pltpu.VMEM((SIZE + 16,), jnp.float32),
            pltpu.SemaphoreType.DMA,
        ],
    )
    def chain(keys_hbm, keys_out, src_out, tgt_out, esrc_out, ew_out,
              kin, newk, S, inclL, inclB, piv_buf, cbuf,
              dest, rows_buf, srcb, ltgt, lsrc, lw, sem):
        wid = lax.axis_index("s") * info.num_cores + lax.axis_index("c")
        IOTA = lax.iota(jnp.int32, 16)

        @pl.when(wid < BATCH)
        def _body():
            b = wid
            boff = b * SIZE
            pltpu.sync_copy(keys_hbm.at[b], kin)

            def seg_scan(v, carry, c, log2seg, segm1):
                base = v * 16
                lanes = IOTA + base
                incl_l = plsc.cumsum(c)
                sstart = lanes - (lanes & segm1)
                sstart_local = jnp.maximum(sstart - base, 0)
                pidx = sstart_local - 1
                g = incl_l[jnp.maximum(pidx, 0)]
                g = jnp.where(pidx >= 0, g, 0)
                incl_f = incl_l - g + jnp.where(sstart < base, carry, 0)
                inclB[pl.ds(base, 16)] = incl_f
                carry_new = incl_f[jnp.full((16,), 15, jnp.int32)]
                return incl_f, carry_new

            def sort_blocks(log2seg):
                seg = lax.shift_left(jnp.int32(1), log2seg)

                @pl.when(log2seg >= 4)
                def _big():
                    def phase0(v, _):
                        S[pl.ds(v * 16, 16)] = jnp.sort(kin[pl.ds(v * 16, 16)])
                        return 0

                    lax.fori_loop(0, NV, phase0, 0)

                    def comparator(u_vr, w_vr):
                        a = S[pl.ds(u_vr * 16, 16)]
                        b = S[pl.ds(w_vr * 16, 16)]
                        br = lax.rev(b, (0,))
                        mn = jnp.minimum(a, br)
                        mx = jnp.maximum(a, br)
                        S[pl.ds(u_vr * 16, 16)] = jnp.sort(mn)
                        S[pl.ds(w_vr * 16, 16)] = jnp.sort(mx)

                    def level(j, _):
                        R = lax.shift_left(jnp.int32(1), j)

                        def mirror(p0, _):
                            for p in (p0, p0 + 32):
                                g = lax.shift_right_logical(p, j)
                                t = p & (R - 1)
                                comparator(g * 2 * R + t,
                                           g * 2 * R + 2 * R - 1 - t)
                            return 0

                        lax.fori_loop(0, 32, mirror, 0)

                        def stage(sd, _):
                            log2d = j - 1 - sd
                            D = lax.shift_left(jnp.int32(1), log2d)

                            def comp(p0, _):
                                for p in (p0, p0 + 32):
                                    u_vr = (lax.shift_left(
                                        lax.shift_right_logical(p, log2d),
                                        log2d + 1)) | (p & (D - 1))
                                    comparator(u_vr, u_vr + D)
                                return 0

                            lax.fori_loop(0, 32, comp, 0)
                            return 0

                        lax.fori_loop(0, j, stage, 0)
                        return 0

                    lax.fori_loop(0, log2seg - 4, level, 0)

                @pl.when(log2seg < 4)
                def _small():
                    def vloop(v, _):
                        base = v * 16

                        def kloop(kk, xc):
                            k = lax.shift_left(jnp.int32(1), kk)

                            def jloop(sd, x2):
                                jv = lax.shift_left(jnp.int32(1),
                                                    kk - 1 - sd)
                                newk[pl.ds(base, 16)] = x2
                                p = plsc.load_gather(
                                    newk, [base + (IOTA ^ jv)])
                                lowlane = (IOTA & jv) == 0
                                asc = ((IOTA & k) == 0) | (k == seg)
                                keep_min = lowlane == asc
                                return jnp.where(keep_min,
                                                 jnp.minimum(x2, p),
                                                 jnp.maximum(x2, p))

                            return lax.fori_loop(0, kk, jloop, xc)

                        xs = lax.fori_loop(1, log2seg + 1, kloop,
                                           kin[pl.ds(base, 16)])
                        S[pl.ds(base, 16)] = xs
                        newk[pl.ds(base, 16)] = jnp.zeros((16,), jnp.float32)
                        return 0

                    lax.fori_loop(0, NV, vloop, 0)

            def init_body(v, _):
                dest[pl.ds(v * 16, 16)] = IOTA + v * 16
                newk[pl.ds(v * 16, 16)] = jnp.zeros((16,), jnp.float32)
                return 0

            lax.fori_loop(0, NV, init_body, 0)

            def layer(d, flag):
                log2seg = 11 - d
                seg = lax.shift_left(jnp.int32(1), log2seg)
                segm1 = seg - 1
                half = lax.shift_right_logical(seg, 1)
                nb = lax.shift_right_logical(jnp.int32(SIZE), log2seg)

                @pl.when(flag > 0)
                def _():
                    sort_blocks(log2seg)

                def piv_body(g, _):
                    j = IOTA + g * 16
                    jc = jnp.minimum(j, nb - 1)
                    i1 = jc * seg + half - 1
                    m1 = plsc.load_gather(S, [i1])
                    m2 = plsc.load_gather(S, [i1 + 1])
                    piv_buf[pl.ds(g * 16, 16)] = (m1 + m2) * 0.5
                    return 0

                lax.fori_loop(0, (nb + 15) >> 4, piv_body, 0)

                def csweep(v, carry):
                    cprev, flagv = carry
                    base = v * 16
                    lanes = IOTA + base
                    kv = kin[pl.ds(base, 16)]
                    pe = plsc.load_gather(
                        piv_buf, [lax.shift_right_logical(lanes, log2seg)])
                    t = (kv - pe) * 10.0
                    c = jnp.where(t >= ZSTAR, jnp.int32(1), jnp.int32(0))
                    cbuf[pl.ds(base, 16)] = c
                    incl_f, carry_new = seg_scan(v, cprev, c, log2seg, segm1)
                    endm = ((lanes + 1) & segm1) == 0
                    degen = endm & (incl_f != half)
                    flagv = flagv | jnp.where(degen, 1, 0)
                    return carry_new, flagv

                zero16 = jnp.zeros((16,), jnp.int32)
                _, flagv = lax.fori_loop(0, NV, csweep, (zero16, zero16))
                flag_new = jnp.where(jnp.max(flagv) > 0,
                                     jnp.int32(1), jnp.int32(0))

                def rsweep(v, _):
                    base = v * 16
                    lanes = IOTA + base
                    c = cbuf[pl.ds(base, 16)]
                    incl_f = inclB[pl.ds(base, 16)]
                    sstart = lanes - (lanes & segm1)
                    ones_b = incl_f - c
                    zeros_b = (lanes - sstart) - ones_b
                    rows = sstart + jnp.where(c == 1, half + ones_b, zeros_b)
                    rows_buf[pl.ds(base, 16)] = rows
                    kv = kin[pl.ds(base, 16)]
                    plsc.addupdate_scatter(newk, [rows], kv, mask=c == 0)
                    plsc.addupdate_scatter(newk, [rows], kv, mask=c == 1)
                    return 0

                lax.fori_loop(0, NV, rsweep, 0)

                def copy_body(v, _):
                    kin[pl.ds(v * 16, 16)] = newk[pl.ds(v * 16, 16)]
                    newk[pl.ds(v * 16, 16)] = jnp.zeros((16,), jnp.float32)
                    dv = dest[pl.ds(v * 16, 16)]
                    dest[pl.ds(v * 16, 16)] = plsc.load_gather(rows_buf, [dv])
                    return 0

                lax.fori_loop(0, NV, copy_body, 0)
                return flag_new

            lax.fori_loop(0, MDEPTH, layer, jnp.int32(1))

            def sinit(v, _):
                srcb[pl.ds(v * 16, 16)] = jnp.full((16,), -1, jnp.int32)
                return 0

            lax.fori_loop(0, NV, sinit, 0)

            def sinv(v, _):
                dv = dest[pl.ds(v * 16, 16)]
                plsc.store_scatter(srcb, [dv], IOTA + v * 16)
                return 0

            lax.fori_loop(0, NV, sinv, 0)

            def linit(v, _):
                ltgt[pl.ds(v * 16, 16)] = jnp.zeros((16,), jnp.int32)
                lsrc[pl.ds(v * 16, 16)] = jnp.zeros((16,), jnp.int32)
                lw[pl.ds(v * 16, 16)] = jnp.zeros((16,), jnp.float32)
                return 0

            lax.fori_loop(0, NV + 1, linit, 0)

            def plan_sweep(v, off):
                base = v * 16
                lanes = IOTA + base
                rep = srcb[pl.ds(base, 16)]
                repc = jnp.clip(rep, 0, SIZE - 1)
                dchk = plsc.load_gather(dest, [repc])
                bad = (rep < 0) | (dchk != lanes)
                srcf = jnp.where(rep < 0, jnp.int32(0), repc)
                rows_buf[pl.ds(base, 16)] = srcf + boff
                plsc.store_compressed(ltgt.at[pl.ds(off, 16)],
                                      lanes + boff, mask=bad)
                plsc.store_compressed(lsrc.at[pl.ds(off, 16)],
                                      srcf + boff, mask=bad)
                plsc.store_compressed(lw.at[pl.ds(off, 16)],
                                      jnp.full((16,), -1.0, jnp.float32),
                                      mask=bad)
                off = off + jnp.max(plsc.all_reduce_population_count(bad))
                dv = dest[pl.ds(base, 16)]
                repi = plsc.load_gather(srcb, [dv])
                ex = repi != lanes
                plsc.store_compressed(ltgt.at[pl.ds(off, 16)], dv + boff, mask=ex)
                plsc.store_compressed(lsrc.at[pl.ds(off, 16)],
                                      lanes + boff, mask=ex)
                plsc.store_compressed(lw.at[pl.ds(off, 16)],
                                      jnp.full((16,), 1.0, jnp.float32),
                                      mask=ex)
                off = off + jnp.max(plsc.all_reduce_population_count(ex))
                return off

            lax.fori_loop(0, NV, plan_sweep, jnp.int32(0))

            pltpu.sync_copy(kin, keys_out.at[b])
            pltpu.sync_copy(rows_buf, src_out.at[b])
            pltpu.sync_copy(ltgt.at[pl.ds(0, EOUT)], tgt_out.at[b])
            pltpu.sync_copy(lsrc.at[pl.ds(0, EOUT)], esrc_out.at[b])
            pltpu.sync_copy(lw.at[pl.ds(0, EOUT)], ew_out.at[b])

    return chain


def _make_apply():
    info = plsc.get_sparse_core_info()
    nw = info.num_cores * info.num_subcores
    rows_per_w = NROWS // nw
    chunk = 512
    nchunks = rows_per_w // chunk
    iw = 128
    nsub = chunk // iw
    mesh = plsc.VectorSubcoreMesh(core_axis_name="c", subcore_axis_name="s")

    @functools.partial(
        pl.kernel, mesh=mesh,
        out_type=jax.ShapeDtypeStruct((NROWS, ZDIM), jnp.float32),
        compiler_params=pltpu.CompilerParams(needs_layout_passes=False),
        scratch_types=[
            pltpu.VMEM((rows_per_w // iw, iw), jnp.int32),
            pltpu.VMEM((chunk, ZDIM), jnp.float32),
            pltpu.VMEM((EOUT,), jnp.int32),
            pltpu.VMEM((EOUT,), jnp.int32),
            pltpu.VMEM((EOUT,), jnp.float32),
            pltpu.VMEM((ECAP, ZDIM), jnp.float32),
            pltpu.SemaphoreType.DMA,
        ],
    )
    def apply_kernel(x_hbm, src_hbm, tgt_hbm, esrc_hbm, ew_hbm, out_hbm,
                     idx_v, rows_v, tgt_v, esrc_v, ew_v, ext_v, sem):
        wid = lax.axis_index("s") * info.num_cores + lax.axis_index("c")
        base = pl.multiple_of(wid * rows_per_w, rows_per_w)
        bt = wid // (SIZE // rows_per_w)
        pltpu.sync_copy(tgt_hbm.at[bt], tgt_v)
        pltpu.sync_copy(esrc_hbm.at[bt], esrc_v)
        pltpu.sync_copy(ew_hbm.at[bt], ew_v)
        pltpu.async_copy(x_hbm.at[esrc_v.at[pl.ds(0, ECAP)]], ext_v, sem).wait()
        pltpu.sync_copy(
            src_hbm.at[pl.ds(pl.multiple_of(base // iw, 8), rows_per_w // iw)],
            idx_v)
        for ci in range(nchunks):
            cbase = pl.multiple_of(base + ci * chunk, chunk)
            copies = [
                pltpu.async_copy(x_hbm.at[idx_v.at[ci * nsub + j]],
                                 rows_v.at[pl.ds(j * iw, iw)], sem)
                for j in range(nsub)
            ]
            for c in copies:
                c.wait()

            cb_vec = jnp.full((16,), cbase, jnp.int32)

            def fix(e, carry):
                e_idx = jnp.full((16,), e, jnp.int32)
                t_vec = plsc.load_gather(tgt_v, [e_idx])
                w_vec = plsc.load_gather(ew_v, [e_idx])
                loc = t_vec - cb_vec
                inside = (loc >= 0) & (loc < chunk)
                wm = jnp.where(inside, w_vec, jnp.zeros((16,), jnp.float32))
                locc = jnp.clip(loc, 0, chunk - 1)
                for kk in range(ZDIM // 16):
                    col = lax.iota(jnp.int32, 16) + (16 * kk)
                    v = plsc.load_gather(ext_v, [e_idx, col])
                    plsc.addupdate_scatter(rows_v, [locc, col], wm * v)
                return carry

            lax.fori_loop(0, ECAP, fix, 0)
            pltpu.sync_copy(rows_v, out_hbm.at[pl.ds(cbase, chunk)])

    return apply_kernel


_chain = _make_chain()
_apply = _make_apply()


def kernel(x, keys):
    out_k, srcg, tgt, esrc, ew = _chain(keys)
    x_flat = x.reshape(NROWS, ZDIM)
    out = _apply(x_flat, srcg.reshape(NROWS // 128, 128), tgt, esrc, ew)
    return out.reshape(x.shape), out_k
